# baseline (speedup 1.0000x reference)
import functools

import jax
import jax.numpy as jnp
from jax import lax
from jax.experimental import pallas as pl
from jax.experimental.pallas import tpu as pltpu
from jax.experimental.pallas import tpu_sc as plsc

CIN = 6
STEM = 64
TOK = 128
NPATCH = 64
RADII = (0.1, 0.2)
K0, K1 = 16, 32
PE_REL = 24
PE_GLB = 96
RW = 16
L = 16
NC, NS = 2, 16
NW = NC * NS

_HI = jax.lax.Precision.HIGHEST


def _dot(a, b):
    return jnp.dot(a, b, precision=_HI, preferred_element_type=jnp.float32)


def _ln(x, g, beta):
    mu = jnp.mean(x, axis=-1, keepdims=True)
    var = jnp.mean((x - mu) ** 2, axis=-1, keepdims=True)
    return (x - mu) / jnp.sqrt(var + 1e-5) * g + beta


def _gelu(x):
    return x * 0.5 * (1.0 + lax.erf(x / jnp.sqrt(2.0).astype(jnp.float32)))


def _mlp(x, w, b, g, beta, act):
    x = _ln(_dot(x, w) + b, g, beta)
    return _gelu(x) if act else x


def _fps_body(xs_ref, ys_ref, zs_ref, cidx_ref, cxo_ref, cyo_ref, czo_ref,
              dist_ref):
    b, n = xs_ref.shape
    lane = lax.broadcasted_iota(jnp.int32, (b, n), 1)
    lane_s = lax.broadcasted_iota(jnp.int32, (b, NPATCH), 1)
    dist_ref[...] = jnp.full((b, n), 1e10, jnp.float32)
    cidx_ref[...] = jnp.zeros((b, NPATCH), jnp.int32)
    cxo_ref[...] = jnp.zeros((b, NPATCH), jnp.float32)
    cyo_ref[...] = jnp.zeros((b, NPATCH), jnp.float32)
    czo_ref[...] = jnp.zeros((b, NPATCH), jnp.float32)

    def step(i, far):
        onehot = (lane_s == i).astype(jnp.int32)
        cidx_ref[...] = cidx_ref[...] + far * onehot
        xs, ys, zs = xs_ref[...], ys_ref[...], zs_ref[...]
        m = (lane == far).astype(jnp.float32)
        cx = jnp.sum(xs * m, axis=1, keepdims=True)
        cy = jnp.sum(ys * m, axis=1, keepdims=True)
        cz = jnp.sum(zs * m, axis=1, keepdims=True)
        onef = onehot.astype(jnp.float32)
        cxo_ref[...] = cxo_ref[...] + cx * onef
        cyo_ref[...] = cyo_ref[...] + cy * onef
        czo_ref[...] = czo_ref[...] + cz * onef
        dx, dy, dz = xs - cx, ys - cy, zs - cz
        d = (dx * dx + dy * dy) + dz * dz
        dist = jnp.minimum(dist_ref[...], d)
        dist_ref[...] = dist
        mx = jnp.max(dist, axis=1, keepdims=True)
        far = jnp.min(jnp.where(dist == mx, lane, n), axis=1,
                      keepdims=True).astype(jnp.int32)
        return far

    far0 = jnp.zeros((b, 1), jnp.int32)
    lax.fori_loop(0, NPATCH, step, far0)


def _fps(xs, ys, zs):
    b, n = xs.shape
    return pl.pallas_call(
        _fps_body,
        out_shape=(jax.ShapeDtypeStruct((b, NPATCH), jnp.int32),
                   jax.ShapeDtypeStruct((b, NPATCH), jnp.float32),
                   jax.ShapeDtypeStruct((b, NPATCH), jnp.float32),
                   jax.ShapeDtypeStruct((b, NPATCH), jnp.float32)),
        scratch_shapes=[pltpu.VMEM((b, n), jnp.float32)],
    )(xs, ys, zs)


SUPER = 2048


def _sc_ball_body(xyzt_hbm, pc_hbm, cidx_hbm, cxyz_hbm,
                  out0_hbm, out1_hbm, outc_hbm,
                  xyz_v, cvi_v, crow_v, cpos_v, idx0_v, idx1_v, buf0_v, buf1_v,
                  rows0_v, rows1_v, cnt_s, sem):
    bsz, _, n = xyzt_hbm.shape
    r0sq = jnp.float32(RADII[0] ** 2)
    r1sq = jnp.float32(RADII[1] ** 2)
    wid = lax.axis_index("c") * NS + lax.axis_index("s")
    bi = wid // 8
    s_base = (wid % 8) * 8
    boff = bi * n

    pltpu.sync_copy(xyzt_hbm.at[bi], xyz_v)
    cvi_v[pl.ds(0, 16)] = jnp.zeros((16,), jnp.int32)
    pltpu.sync_copy(cidx_hbm.at[bi, pl.ds(s_base, 8)], cvi_v.at[pl.ds(0, 8)])
    pltpu.async_copy(pc_hbm.at[cvi_v], crow_v, sem).wait()
    pltpu.sync_copy(crow_v.at[pl.ds(0, 8)], outc_hbm.at[bi, pl.ds(s_base, 8)])
    pltpu.sync_copy(cxyz_hbm.at[bi], cpos_v.at[pl.ds(0, 3 * NPATCH)])

    io = lax.iota(jnp.int32, 16)
    iof = io.astype(jnp.float32)
    tgtf = iof + 1.0

    def append(m, slot, cap, buf, pidxf):
        c = cnt_s[slot]
        s = jnp.where(m, 1.0, 0.0)
        for dd in (1, 2, 4, 8):
            s = s + jnp.where(io >= dd, s[jnp.maximum(io - dd, 0)], 0.0)
        pc = s[15].astype(jnp.int32)
        pos = jnp.zeros((16,), jnp.int32)
        for dd in (8, 4, 2, 1):
            pr = s[jnp.minimum(pos + (dd - 1), 15)]
            pos = jnp.where(pr < tgtf, pos + dd, pos)
        buf[pl.ds(c, 16)] = pidxf[pos]
        cnt_s[slot] = jnp.minimum(c + pc, cap)

    def center_body(j, _unused):
        sj = s_base + j
        cx = cpos_v[pl.ds(sj, 16)][0]
        cy = cpos_v[pl.ds(NPATCH + sj, 16)][0]
        cz = cpos_v[pl.ds(2 * NPATCH + sj, 16)][0]
        cnt_s[0] = 0
        cnt_s[1] = 0

        def outer(sb, _):
            c0o = cnt_s[0]
            c1o = cnt_s[1]

            @pl.when((c0o < K0) | (c1o < K1))
            def _():
                @pl.loop(0, SUPER // (4 * L))
                def _(ch):
                    base = sb * SUPER + ch * (4 * L)
                    ds, dmins = [], []
                    for q in range(4):
                        off = base + q * L
                        xv = xyz_v[0, pl.ds(off, L)]
                        yv = xyz_v[1, pl.ds(off, L)]
                        zv = xyz_v[2, pl.ds(off, L)]
                        dx, dy, dz = xv - cx, yv - cy, zv - cz
                        ds.append((dx * dx + dy * dy) + dz * dz)
                    for q in range(4):
                        dm = ds[q]
                        for dd in (8, 4, 2, 1):
                            dm = jnp.minimum(dm, dm[io ^ dd])
                        dmins.append(dm[0])
                    dmin = jnp.minimum(jnp.minimum(dmins[0], dmins[1]),
                                       jnp.minimum(dmins[2], dmins[3]))
                    hit1 = (dmin <= r1sq) & (cnt_s[1] < K1)
                    hit0 = (dmin <= r0sq) & (cnt_s[0] < K0)

                    @pl.when(hit1 | hit0)
                    def _():
                        for q in range(4):
                            off = base + q * L
                            d = ds[q]
                            pidxf = (jnp.float32(boff)
                                     + (off + io).astype(jnp.float32))

                            @pl.when((dmins[q] <= r1sq) & (cnt_s[1] < K1))
                            def _():
                                append(d <= r1sq, 1, K1, buf1_v, pidxf)

                            @pl.when((dmins[q] <= r0sq) & (cnt_s[0] < K0))
                            def _():
                                append(d <= r0sq, 0, K0, buf0_v, pidxf)

            return 0

        lax.fori_loop(0, n // SUPER, outer, 0)

        c0f = cnt_s[0]
        c1f = cnt_s[1]
        v0 = buf0_v[pl.ds(0, 16)].astype(jnp.int32)
        first0 = v0[0]
        idx0_v[pl.ds(0, 16)] = jnp.where(io < c0f, v0, first0)
        first1 = buf1_v[pl.ds(0, 16)].astype(jnp.int32)[0]
        for h in range(2):
            vh = buf1_v[pl.ds(h * 16, 16)].astype(jnp.int32)
            gi = (h * 16) + io
            idx1_v[pl.ds(h * 16, 16)] = jnp.where(gi < c1f, vh, first1)

        pltpu.async_copy(pc_hbm.at[idx0_v], rows0_v, sem).wait()
        pltpu.async_copy(pc_hbm.at[idx1_v], rows1_v, sem).wait()
        pltpu.sync_copy(rows0_v, out0_hbm.at[bi, s_base + j])
        pltpu.sync_copy(rows1_v, out1_hbm.at[bi, s_base + j])
        return 0

    lax.fori_loop(0, 8, center_body, 0)


def _sc_ball(xyzt, pc_flat, cidx_adj, cxyz):
    bsz, _, n = xyzt.shape
    mesh = plsc.VectorSubcoreMesh(core_axis_name="c", subcore_axis_name="s",
                                  num_cores=NC, num_subcores=NS)
    f = pl.kernel(
        _sc_ball_body,
        out_type=(jax.ShapeDtypeStruct((bsz, NPATCH, K0, CIN), jnp.float32),
                  jax.ShapeDtypeStruct((bsz, NPATCH, K1, CIN), jnp.float32),
                  jax.ShapeDtypeStruct((bsz, NPATCH, CIN), jnp.float32)),
        mesh=mesh,
        scratch_types=[
            pltpu.VMEM((3, n), jnp.float32),
            pltpu.VMEM((16,), jnp.int32),
            pltpu.VMEM((16, CIN), jnp.float32),
            pltpu.VMEM((3 * NPATCH + 16,), jnp.float32),
            pltpu.VMEM((K0,), jnp.int32),
            pltpu.VMEM((K1,), jnp.int32),
            pltpu.VMEM((K0 + 16,), jnp.float32),
            pltpu.VMEM((K1 + 16,), jnp.float32),
            pltpu.VMEM((K0, CIN), jnp.float32),
            pltpu.VMEM((K1, CIN), jnp.float32),
            pltpu.SMEM((4,), jnp.int32),
            pltpu.SemaphoreType.DMA,
        ],
        compiler_params=pltpu.CompilerParams(use_tc_tiling_on_sc=False),
    )
    return f(xyzt, pc_flat, cidx_adj, cxyz)


def _scale_body(x_ref, c_ref, w1, b1, g1, be1, w2, b2, g2, be2,
                e24_ref, wgh, wgr, wgp, bg, gg, beg, wh, bh, gh, beh,
                out_ref, *, k, inv_r):
    x = x_ref[...]
    h = _mlp(x, w1[...], b1[...], g1[...], be1[...], True)
    h = _mlp(h, w2[...], b2[...], g2[...], be2[...], True)
    r8 = x.shape[0]
    colm = (lax.broadcasted_iota(jnp.int32, (1, 8), 1) < 3).astype(jnp.float32)
    relp = (x[:, 0:8] - c_ref[...][:, 0:8]) * jnp.float32(inv_r) * colm
    args = _dot(relp, e24_ref[...])
    sel = (lax.broadcasted_iota(jnp.int32, (r8, PE_REL), 1) % 8) < 4
    pe = jnp.where(sel, jnp.sin(args), jnp.cos(args))
    z = _dot(h, wgh[...]) + _dot(relp, wgr[...]) + _dot(pe, wgp[...]) + bg[...]
    z = _gelu(_ln(z, gg[...], beg[...]))
    z = _mlp(z, wh[...], bh[...], gh[...], beh[...], False)
    grp = z.reshape(r8 // k, k, TOK)
    out_ref[...] = jnp.max(grp, axis=1)


def _scale(rows, crep, p_stem, enc, k, radius):
    r8 = rows.shape[0]
    w1 = jnp.zeros((RW, STEM), jnp.float32).at[:CIN].set(p_stem[0]["W"])
    wg = enc[0]["W"]
    wgh = wg[:STEM]
    wgr = jnp.zeros((8, TOK), jnp.float32).at[:3].set(wg[STEM:STEM + 3])
    wgp = wg[STEM + 3:]
    freqs = (2.0 ** jnp.arange(4, dtype=jnp.float32)) * jnp.pi
    e24 = jnp.zeros((8, PE_REL), jnp.float32)
    for c in range(3):
        e24 = e24.at[c, c * 8:c * 8 + 4].set(freqs)
        e24 = e24.at[c, c * 8 + 4:c * 8 + 8].set(freqs)
    row = lambda a: a.reshape(1, -1)
    body = functools.partial(_scale_body, k=k, inv_r=1.0 / max(radius, 1e-6))
    return pl.pallas_call(
        body,
        out_shape=jax.ShapeDtypeStruct((r8 // k, TOK), jnp.float32),
    )(rows, crep,
      w1, row(p_stem[0]["b"]), row(p_stem[0]["g"]), row(p_stem[0]["beta"]),
      p_stem[1]["W"], row(p_stem[1]["b"]), row(p_stem[1]["g"]), row(p_stem[1]["beta"]),
      e24, wgh, wgr, wgp,
      row(enc[0]["b"]), row(enc[0]["g"]), row(enc[0]["beta"]),
      enc[1]["W"], row(enc[1]["b"]), row(enc[1]["g"]), row(enc[1]["beta"]))


def _proj_body(c_ref, s0_ref, s1_ref, w1, b1, g1, be1, w2, b2, g2, be2,
               e96_ref, wpc, wp0, wp1, wpe, bp, gp, bep,
               wq, bq, gq, beq, out_ref):
    x = c_ref[...]
    cf = _mlp(x, w1[...], b1[...], g1[...], be1[...], True)
    cf = _mlp(cf, w2[...], b2[...], g2[...], be2[...], True)
    r8 = x.shape[0]
    colm = (lax.broadcasted_iota(jnp.int32, (1, 8), 1) < 3).astype(jnp.float32)
    cxyz = x[:, 0:8] * colm
    args = _dot(cxyz, e96_ref[...])
    sel = (lax.broadcasted_iota(jnp.int32, (r8, PE_GLB), 1) % 32) < 16
    cpe = jnp.where(sel, jnp.sin(args), jnp.cos(args))
    z = (_dot(cf, wpc[...]) + _dot(s0_ref[...], wp0[...])
         + _dot(s1_ref[...], wp1[...]) + _dot(cpe, wpe[...]) + bp[...])
    z = _gelu(_ln(z, gp[...], bep[...]))
    z = _mlp(z, wq[...], bq[...], gq[...], beq[...], False)
    out_ref[...] = z


def _proj(rows_c, s0, s1, p_stem, proj):
    bs = rows_c.shape[0]
    w1 = jnp.zeros((RW, STEM), jnp.float32).at[:CIN].set(p_stem[0]["W"])
    wp = proj[0]["W"]
    wpc = wp[:STEM]
    wp0 = wp[STEM:STEM + TOK]
    wp1 = wp[STEM + TOK:STEM + 2 * TOK]
    wpe = wp[STEM + 2 * TOK:]
    freqs = (2.0 ** jnp.arange(16, dtype=jnp.float32)) * jnp.pi
    e96 = jnp.zeros((8, PE_GLB), jnp.float32)
    for c in range(3):
        e96 = e96.at[c, c * 32:c * 32 + 16].set(freqs)
        e96 = e96.at[c, c * 32 + 16:c * 32 + 32].set(freqs)
    row = lambda a: a.reshape(1, -1)
    return pl.pallas_call(
        _proj_body,
        out_shape=jax.ShapeDtypeStruct((bs, TOK), jnp.float32),
    )(rows_c, s0, s1,
      w1, row(p_stem[0]["b"]), row(p_stem[0]["g"]), row(p_stem[0]["beta"]),
      p_stem[1]["W"], row(p_stem[1]["b"]), row(p_stem[1]["g"]), row(p_stem[1]["beta"]),
      e96, wpc, wp0, wp1, wpe,
      row(proj[0]["b"]), row(proj[0]["g"]), row(proj[0]["beta"]),
      proj[1]["W"], row(proj[1]["b"]), row(proj[1]["g"]), row(proj[1]["beta"]))


def kernel(pointcloud, params):
    b, n, cin = pointcloud.shape
    xyzt = jnp.transpose(pointcloud[..., :3], (0, 2, 1))
    cidx, cxo, cyo, czo = _fps(xyzt[:, 0], xyzt[:, 1], xyzt[:, 2])
    cxyz = jnp.concatenate([cxo, cyo, czo], axis=1)
    pc_flat = pointcloud.reshape(b * n, cin)
    cidx_adj = cidx + (jnp.arange(b, dtype=jnp.int32) * n)[:, None]
    rows0, rows1, rows_c = _sc_ball(xyzt, pc_flat, cidx_adj, cxyz)
    centers = rows_c[..., :3]
    bs = b * NPATCH
    pad = ((0, 0), (0, RW - cin))
    rows0p = jnp.pad(rows0.reshape(bs * K0, cin), pad)
    rows1p = jnp.pad(rows1.reshape(bs * K1, cin), pad)
    rows_cp = jnp.pad(rows_c.reshape(bs, cin), pad)
    crep = rows_cp.reshape(b, NPATCH, 1, RW)
    s0 = _scale(rows0p,
                jnp.broadcast_to(crep, (b, NPATCH, K0, RW)).reshape(bs * K0, RW),
                params["stem"], params["enc"][0], K0, RADII[0])
    s1 = _scale(rows1p,
                jnp.broadcast_to(crep, (b, NPATCH, K1, RW)).reshape(bs * K1, RW),
                params["stem"], params["enc"][1], K1, RADII[1])
    t = _proj(rows_cp, s0, s1, params["stem"], params["proj"])
    return t.reshape(b, NPATCH, TOK), centers


# speedup vs baseline: 1.0135x; 1.0135x over previous
import functools

import jax
import jax.numpy as jnp
from jax import lax
from jax.experimental import pallas as pl
from jax.experimental.pallas import tpu as pltpu
from jax.experimental.pallas import tpu_sc as plsc

CIN = 6
STEM = 64
TOK = 128
NPATCH = 64
RADII = (0.1, 0.2)
K0, K1 = 16, 32
PE_REL = 24
PE_GLB = 96
RW = 16
L = 16
NC, NS = 2, 16
NW = NC * NS

_HI = jax.lax.Precision.HIGHEST


def _dot(a, b):
    return jnp.dot(a, b, precision=_HI, preferred_element_type=jnp.float32)


def _ln(x, g, beta):
    mu = jnp.mean(x, axis=-1, keepdims=True)
    var = jnp.mean((x - mu) ** 2, axis=-1, keepdims=True)
    return (x - mu) / jnp.sqrt(var + 1e-5) * g + beta


def _gelu(x):
    return x * 0.5 * (1.0 + lax.erf(x / jnp.sqrt(2.0).astype(jnp.float32)))


def _mlp(x, w, b, g, beta, act):
    x = _ln(_dot(x, w) + b, g, beta)
    return _gelu(x) if act else x


def _fps_body(xs_ref, ys_ref, zs_ref, cidx_ref, dist_ref):
    b, n = xs_ref.shape
    lane = lax.broadcasted_iota(jnp.int32, (b, n), 1)
    lane_s = lax.broadcasted_iota(jnp.int32, (b, NPATCH), 1)
    dist_ref[...] = jnp.full((b, n), 1e10, jnp.float32)
    cidx_ref[...] = jnp.zeros((b, NPATCH), jnp.int32)

    def step(i, far):
        onehot = (lane_s == i).astype(jnp.int32)
        cidx_ref[...] = cidx_ref[...] + far * onehot
        xs, ys, zs = xs_ref[...], ys_ref[...], zs_ref[...]
        m = (lane == far).astype(jnp.float32)
        cx = jnp.sum(xs * m, axis=1, keepdims=True)
        cy = jnp.sum(ys * m, axis=1, keepdims=True)
        cz = jnp.sum(zs * m, axis=1, keepdims=True)
        dx, dy, dz = xs - cx, ys - cy, zs - cz
        d = (dx * dx + dy * dy) + dz * dz
        dist = jnp.minimum(dist_ref[...], d)
        dist_ref[...] = dist
        mx = jnp.max(dist, axis=1, keepdims=True)
        far = jnp.min(jnp.where(dist == mx, lane, n), axis=1,
                      keepdims=True).astype(jnp.int32)
        return far

    far0 = jnp.zeros((b, 1), jnp.int32)
    lax.fori_loop(0, NPATCH, step, far0)


def _fps(xs, ys, zs):
    b, n = xs.shape
    return pl.pallas_call(
        _fps_body,
        out_shape=jax.ShapeDtypeStruct((b, NPATCH), jnp.int32),
        scratch_shapes=[pltpu.VMEM((b, n), jnp.float32)],
    )(xs, ys, zs)


SUPER = 2048


def _sc_ball_body(xyzt_hbm, pc_hbm, cidx_hbm,
                  out0_hbm, out1_hbm, outc_hbm,
                  xyz_v, cvi_v, crow_v, idx0_v, idx1_v, buf0_v, buf1_v,
                  rows0_v, rows1_v, cnt_s, sem):
    bsz, _, n = xyzt_hbm.shape
    r0sq = jnp.float32(RADII[0] ** 2)
    r1sq = jnp.float32(RADII[1] ** 2)
    wid = lax.axis_index("c") * NS + lax.axis_index("s")
    bi = wid // 8
    s_base = (wid % 8) * 8
    boff = bi * n

    pltpu.sync_copy(xyzt_hbm.at[bi], xyz_v)
    pltpu.sync_copy(cidx_hbm.at[bi, pl.ds(s_base, 8)], cvi_v)
    pltpu.async_copy(pc_hbm.at[cvi_v], crow_v, sem).wait()
    pltpu.sync_copy(crow_v, outc_hbm.at[bi, pl.ds(s_base, 8)])

    io = lax.iota(jnp.int32, 16)
    iof = io.astype(jnp.float32)
    tgtf = iof + 1.0

    def append(m, slot, cap, buf, pidxf):
        c = cnt_s[slot]
        s = jnp.where(m, 1.0, 0.0)
        for dd in (1, 2, 4, 8):
            s = s + jnp.where(io >= dd, s[jnp.maximum(io - dd, 0)], 0.0)
        pc = s[15].astype(jnp.int32)
        pos = jnp.zeros((16,), jnp.int32)
        for dd in (8, 4, 2, 1):
            pr = s[jnp.minimum(pos + (dd - 1), 15)]
            pos = jnp.where(pr < tgtf, pos + dd, pos)
        buf[pl.ds(c, 16)] = pidxf[pos]
        cnt_s[slot] = jnp.minimum(c + pc, cap)

    def center_body(j, _unused):
        crow_j = crow_v[j, pl.ds(0, 16)]
        cx = crow_j[0]
        cy = crow_j[1]
        cz = crow_j[2]
        cnt_s[0] = 0
        cnt_s[1] = 0

        def outer(sb, _):
            c0o = cnt_s[0]
            c1o = cnt_s[1]

            @pl.when((c0o < K0) | (c1o < K1))
            def _():
                @pl.loop(0, SUPER // (4 * L))
                def _(ch):
                    base = sb * SUPER + ch * (4 * L)
                    ds, dmins = [], []
                    for q in range(4):
                        off = base + q * L
                        xv = xyz_v[0, pl.ds(off, L)]
                        yv = xyz_v[1, pl.ds(off, L)]
                        zv = xyz_v[2, pl.ds(off, L)]
                        dx, dy, dz = xv - cx, yv - cy, zv - cz
                        ds.append((dx * dx + dy * dy) + dz * dz)
                    for q in range(4):
                        dm = ds[q]
                        for dd in (8, 4, 2, 1):
                            dm = jnp.minimum(dm, dm[io ^ dd])
                        dmins.append(dm[0])
                    dmin = jnp.minimum(jnp.minimum(dmins[0], dmins[1]),
                                       jnp.minimum(dmins[2], dmins[3]))
                    hit1 = (dmin <= r1sq) & (cnt_s[1] < K1)
                    hit0 = (dmin <= r0sq) & (cnt_s[0] < K0)

                    @pl.when(hit1 | hit0)
                    def _():
                        for q in range(4):
                            off = base + q * L
                            d = ds[q]
                            pidxf = (jnp.float32(boff)
                                     + (off + io).astype(jnp.float32))

                            @pl.when((dmins[q] <= r1sq) & (cnt_s[1] < K1))
                            def _():
                                append(d <= r1sq, 1, K1, buf1_v, pidxf)

                            @pl.when((dmins[q] <= r0sq) & (cnt_s[0] < K0))
                            def _():
                                append(d <= r0sq, 0, K0, buf0_v, pidxf)

            return 0

        lax.fori_loop(0, n // SUPER, outer, 0)

        c0f = cnt_s[0]
        c1f = cnt_s[1]
        v0 = buf0_v[pl.ds(0, 16)].astype(jnp.int32)
        first0 = v0[0]
        idx0_v[pl.ds(0, 16)] = jnp.where(io < c0f, v0, first0)
        first1 = buf1_v[pl.ds(0, 16)].astype(jnp.int32)[0]
        for h in range(2):
            vh = buf1_v[pl.ds(h * 16, 16)].astype(jnp.int32)
            gi = (h * 16) + io
            idx1_v[pl.ds(h * 16, 16)] = jnp.where(gi < c1f, vh, first1)

        pltpu.async_copy(pc_hbm.at[idx0_v], rows0_v, sem).wait()
        pltpu.async_copy(pc_hbm.at[idx1_v], rows1_v, sem).wait()
        pltpu.sync_copy(rows0_v, out0_hbm.at[bi, s_base + j])
        pltpu.sync_copy(rows1_v, out1_hbm.at[bi, s_base + j])
        return 0

    lax.fori_loop(0, 8, center_body, 0)


def _sc_ball(xyzt, pc_flat, cidx_adj):
    bsz, _, n = xyzt.shape
    mesh = plsc.VectorSubcoreMesh(core_axis_name="c", subcore_axis_name="s",
                                  num_cores=NC, num_subcores=NS)
    f = pl.kernel(
        _sc_ball_body,
        out_type=(jax.ShapeDtypeStruct((bsz, NPATCH, K0, RW), jnp.float32),
                  jax.ShapeDtypeStruct((bsz, NPATCH, K1, RW), jnp.float32),
                  jax.ShapeDtypeStruct((bsz, NPATCH, RW), jnp.float32)),
        mesh=mesh,
        scratch_types=[
            pltpu.VMEM((3, n), jnp.float32),
            pltpu.VMEM((8,), jnp.int32),
            pltpu.VMEM((8, RW), jnp.float32),
            pltpu.VMEM((K0,), jnp.int32),
            pltpu.VMEM((K1,), jnp.int32),
            pltpu.VMEM((K0 + 16,), jnp.float32),
            pltpu.VMEM((K1 + 16,), jnp.float32),
            pltpu.VMEM((K0, RW), jnp.float32),
            pltpu.VMEM((K1, RW), jnp.float32),
            pltpu.SMEM((4,), jnp.int32),
            pltpu.SemaphoreType.DMA,
        ],
        compiler_params=pltpu.CompilerParams(use_tc_tiling_on_sc=False),
    )
    return f(xyzt, pc_flat, cidx_adj)


def _scale_body(x_ref, c_ref, w1, b1, g1, be1, w2, b2, g2, be2,
                e24_ref, wgh, wgr, wgp, bg, gg, beg, wh, bh, gh, beh,
                out_ref, *, k, inv_r):
    x = x_ref[...]
    h = _mlp(x, w1[...], b1[...], g1[...], be1[...], True)
    h = _mlp(h, w2[...], b2[...], g2[...], be2[...], True)
    r8 = x.shape[0]
    colm = (lax.broadcasted_iota(jnp.int32, (1, 8), 1) < 3).astype(jnp.float32)
    relp = (x[:, 0:8] - c_ref[...][:, 0:8]) * jnp.float32(inv_r) * colm
    args = _dot(relp, e24_ref[...])
    sel = (lax.broadcasted_iota(jnp.int32, (r8, PE_REL), 1) % 8) < 4
    pe = jnp.where(sel, jnp.sin(args), jnp.cos(args))
    z = _dot(h, wgh[...]) + _dot(relp, wgr[...]) + _dot(pe, wgp[...]) + bg[...]
    z = _gelu(_ln(z, gg[...], beg[...]))
    z = _mlp(z, wh[...], bh[...], gh[...], beh[...], False)
    grp = z.reshape(r8 // k, k, TOK)
    out_ref[...] = jnp.max(grp, axis=1)


def _scale(rows, crep, p_stem, enc, k, radius):
    r8 = rows.shape[0]
    w1 = jnp.zeros((RW, STEM), jnp.float32).at[:CIN].set(p_stem[0]["W"])
    wg = enc[0]["W"]
    wgh = wg[:STEM]
    wgr = jnp.zeros((8, TOK), jnp.float32).at[:3].set(wg[STEM:STEM + 3])
    wgp = wg[STEM + 3:]
    freqs = (2.0 ** jnp.arange(4, dtype=jnp.float32)) * jnp.pi
    e24 = jnp.zeros((8, PE_REL), jnp.float32)
    for c in range(3):
        e24 = e24.at[c, c * 8:c * 8 + 4].set(freqs)
        e24 = e24.at[c, c * 8 + 4:c * 8 + 8].set(freqs)
    row = lambda a: a.reshape(1, -1)
    body = functools.partial(_scale_body, k=k, inv_r=1.0 / max(radius, 1e-6))
    return pl.pallas_call(
        body,
        out_shape=jax.ShapeDtypeStruct((r8 // k, TOK), jnp.float32),
    )(rows, crep,
      w1, row(p_stem[0]["b"]), row(p_stem[0]["g"]), row(p_stem[0]["beta"]),
      p_stem[1]["W"], row(p_stem[1]["b"]), row(p_stem[1]["g"]), row(p_stem[1]["beta"]),
      e24, wgh, wgr, wgp,
      row(enc[0]["b"]), row(enc[0]["g"]), row(enc[0]["beta"]),
      enc[1]["W"], row(enc[1]["b"]), row(enc[1]["g"]), row(enc[1]["beta"]))


def _proj_body(c_ref, s0_ref, s1_ref, w1, b1, g1, be1, w2, b2, g2, be2,
               e96_ref, wpc, wp0, wp1, wpe, bp, gp, bep,
               wq, bq, gq, beq, out_ref):
    x = c_ref[...]
    cf = _mlp(x, w1[...], b1[...], g1[...], be1[...], True)
    cf = _mlp(cf, w2[...], b2[...], g2[...], be2[...], True)
    r8 = x.shape[0]
    colm = (lax.broadcasted_iota(jnp.int32, (1, 8), 1) < 3).astype(jnp.float32)
    cxyz = x[:, 0:8] * colm
    args = _dot(cxyz, e96_ref[...])
    sel = (lax.broadcasted_iota(jnp.int32, (r8, PE_GLB), 1) % 32) < 16
    cpe = jnp.where(sel, jnp.sin(args), jnp.cos(args))
    z = (_dot(cf, wpc[...]) + _dot(s0_ref[...], wp0[...])
         + _dot(s1_ref[...], wp1[...]) + _dot(cpe, wpe[...]) + bp[...])
    z = _gelu(_ln(z, gp[...], bep[...]))
    z = _mlp(z, wq[...], bq[...], gq[...], beq[...], False)
    out_ref[...] = z


def _proj(rows_c, s0, s1, p_stem, proj):
    bs = rows_c.shape[0]
    w1 = jnp.zeros((RW, STEM), jnp.float32).at[:CIN].set(p_stem[0]["W"])
    wp = proj[0]["W"]
    wpc = wp[:STEM]
    wp0 = wp[STEM:STEM + TOK]
    wp1 = wp[STEM + TOK:STEM + 2 * TOK]
    wpe = wp[STEM + 2 * TOK:]
    freqs = (2.0 ** jnp.arange(16, dtype=jnp.float32)) * jnp.pi
    e96 = jnp.zeros((8, PE_GLB), jnp.float32)
    for c in range(3):
        e96 = e96.at[c, c * 32:c * 32 + 16].set(freqs)
        e96 = e96.at[c, c * 32 + 16:c * 32 + 32].set(freqs)
    row = lambda a: a.reshape(1, -1)
    return pl.pallas_call(
        _proj_body,
        out_shape=jax.ShapeDtypeStruct((bs, TOK), jnp.float32),
    )(rows_c, s0, s1,
      w1, row(p_stem[0]["b"]), row(p_stem[0]["g"]), row(p_stem[0]["beta"]),
      p_stem[1]["W"], row(p_stem[1]["b"]), row(p_stem[1]["g"]), row(p_stem[1]["beta"]),
      e96, wpc, wp0, wp1, wpe,
      row(proj[0]["b"]), row(proj[0]["g"]), row(proj[0]["beta"]),
      proj[1]["W"], row(proj[1]["b"]), row(proj[1]["g"]), row(proj[1]["beta"]))


def kernel(pointcloud, params):
    b, n, cin = pointcloud.shape
    xyzt = jnp.transpose(pointcloud[..., :3], (0, 2, 1))
    cidx = _fps(xyzt[:, 0], xyzt[:, 1], xyzt[:, 2])
    pc_flat = jnp.pad(pointcloud, ((0, 0), (0, 0), (0, RW - cin)))
    pc_flat = pc_flat.reshape(b * n, RW)
    cidx_adj = cidx + (jnp.arange(b, dtype=jnp.int32) * n)[:, None]
    rows0, rows1, rows_c = _sc_ball(xyzt, pc_flat, cidx_adj)
    centers = rows_c[..., :3]
    bs = b * NPATCH
    crep = rows_c.reshape(b, NPATCH, 1, RW)
    s0 = _scale(rows0.reshape(bs * K0, RW),
                jnp.broadcast_to(crep, (b, NPATCH, K0, RW)).reshape(bs * K0, RW),
                params["stem"], params["enc"][0], K0, RADII[0])
    s1 = _scale(rows1.reshape(bs * K1, RW),
                jnp.broadcast_to(crep, (b, NPATCH, K1, RW)).reshape(bs * K1, RW),
                params["stem"], params["enc"][1], K1, RADII[1])
    t = _proj(rows_c.reshape(bs, RW), s0, s1, params["stem"], params["proj"])
    return t.reshape(b, NPATCH, TOK), centers


# speedup vs baseline: 1.1037x; 1.0890x over previous
import functools

import jax
import jax.numpy as jnp
from jax import lax
from jax.experimental import pallas as pl
from jax.experimental.pallas import tpu as pltpu
from jax.experimental.pallas import tpu_sc as plsc

CIN = 6
STEM = 64
TOK = 128
NPATCH = 64
RADII = (0.1, 0.2)
K0, K1 = 16, 32
PE_REL = 24
PE_GLB = 96
RW = 16
L = 16
NC, NS = 2, 16
NW = NC * NS

def _dot(a, b):
    return jnp.dot(a, b, preferred_element_type=jnp.float32)


def _ln(x, g, beta):
    mu = jnp.mean(x, axis=-1, keepdims=True)
    var = jnp.mean((x - mu) ** 2, axis=-1, keepdims=True)
    return (x - mu) / jnp.sqrt(var + 1e-5) * g + beta


def _gelu(x):
    return x * 0.5 * (1.0 + lax.erf(x / jnp.sqrt(2.0).astype(jnp.float32)))


def _mlp(x, w, b, g, beta, act):
    x = _ln(_dot(x, w) + b, g, beta)
    return _gelu(x) if act else x


def _fps_body(xs_ref, ys_ref, zs_ref, cidx_ref, dist_ref):
    b, n = xs_ref.shape
    lane = lax.broadcasted_iota(jnp.int32, (b, n), 1)
    lane_s = lax.broadcasted_iota(jnp.int32, (b, NPATCH), 1)
    dist_ref[...] = jnp.full((b, n), 1e10, jnp.float32)
    cidx_ref[...] = jnp.zeros((b, NPATCH), jnp.int32)

    def step(i, far):
        onehot = (lane_s == i).astype(jnp.int32)
        cidx_ref[...] = cidx_ref[...] + far * onehot
        xs, ys, zs = xs_ref[...], ys_ref[...], zs_ref[...]
        m = (lane == far).astype(jnp.float32)
        cx = jnp.sum(xs * m, axis=1, keepdims=True)
        cy = jnp.sum(ys * m, axis=1, keepdims=True)
        cz = jnp.sum(zs * m, axis=1, keepdims=True)
        dx, dy, dz = xs - cx, ys - cy, zs - cz
        d = (dx * dx + dy * dy) + dz * dz
        dist = jnp.minimum(dist_ref[...], d)
        dist_ref[...] = dist
        mx = jnp.max(dist, axis=1, keepdims=True)
        far = jnp.min(jnp.where(dist == mx, lane, n), axis=1,
                      keepdims=True).astype(jnp.int32)
        return far

    far0 = jnp.zeros((b, 1), jnp.int32)
    lax.fori_loop(0, NPATCH, step, far0)


def _fps(xs, ys, zs):
    b, n = xs.shape
    return pl.pallas_call(
        _fps_body,
        out_shape=jax.ShapeDtypeStruct((b, NPATCH), jnp.int32),
        scratch_shapes=[pltpu.VMEM((b, n), jnp.float32)],
    )(xs, ys, zs)


SUPER = 2048


def _sc_ball_body(xyzt_hbm, pc_hbm, cidx_hbm,
                  out0_hbm, out1_hbm, outc_hbm,
                  xyz_v, cvi_v, crow_v, idx0_v, idx1_v, buf0_v, buf1_v,
                  rows0_v, rows1_v, cnt_s, sem):
    bsz, _, n = xyzt_hbm.shape
    r0sq = jnp.float32(RADII[0] ** 2)
    r1sq = jnp.float32(RADII[1] ** 2)
    wid = lax.axis_index("c") * NS + lax.axis_index("s")
    bi = wid // 8
    s_base = (wid % 8) * 8
    boff = bi * n

    pltpu.sync_copy(xyzt_hbm.at[bi], xyz_v)
    pltpu.sync_copy(cidx_hbm.at[bi, pl.ds(s_base, 8)], cvi_v)
    pltpu.async_copy(pc_hbm.at[cvi_v], crow_v, sem).wait()
    pltpu.sync_copy(crow_v, outc_hbm.at[bi, pl.ds(s_base, 8)])

    io = lax.iota(jnp.int32, 16)
    iof = io.astype(jnp.float32)
    tgtf = iof + 1.0

    def append(m, slot, cap, buf, pidxf):
        c = cnt_s[slot]
        s = jnp.where(m, 1.0, 0.0)
        for dd in (1, 2, 4, 8):
            s = s + jnp.where(io >= dd, s[jnp.maximum(io - dd, 0)], 0.0)
        pc = s[15].astype(jnp.int32)
        pos = jnp.zeros((16,), jnp.int32)
        for dd in (8, 4, 2, 1):
            pr = s[jnp.minimum(pos + (dd - 1), 15)]
            pos = jnp.where(pr < tgtf, pos + dd, pos)
        buf[pl.ds(c, 16)] = pidxf[pos]
        cnt_s[slot] = jnp.minimum(c + pc, cap)

    def center_body(j, _unused):
        crow_j = crow_v[j, pl.ds(0, 16)]
        cx = crow_j[0]
        cy = crow_j[1]
        cz = crow_j[2]
        cnt_s[0] = 0
        cnt_s[1] = 0

        def outer(sb, _):
            c0o = cnt_s[0]
            c1o = cnt_s[1]

            @pl.when((c0o < K0) | (c1o < K1))
            def _():
                @pl.loop(0, SUPER // (4 * L))
                def _(ch):
                    base = sb * SUPER + ch * (4 * L)
                    ds, dmins = [], []
                    for q in range(4):
                        off = base + q * L
                        xv = xyz_v[0, pl.ds(off, L)]
                        yv = xyz_v[1, pl.ds(off, L)]
                        zv = xyz_v[2, pl.ds(off, L)]
                        dx, dy, dz = xv - cx, yv - cy, zv - cz
                        ds.append((dx * dx + dy * dy) + dz * dz)
                    for q in range(4):
                        dm = ds[q]
                        for dd in (8, 4, 2, 1):
                            dm = jnp.minimum(dm, dm[io ^ dd])
                        dmins.append(dm[0])
                    dmin = jnp.minimum(jnp.minimum(dmins[0], dmins[1]),
                                       jnp.minimum(dmins[2], dmins[3]))
                    hit1 = (dmin <= r1sq) & (cnt_s[1] < K1)
                    hit0 = (dmin <= r0sq) & (cnt_s[0] < K0)

                    @pl.when(hit1 | hit0)
                    def _():
                        for q in range(4):
                            off = base + q * L
                            d = ds[q]
                            pidxf = (jnp.float32(boff)
                                     + (off + io).astype(jnp.float32))

                            @pl.when((dmins[q] <= r1sq) & (cnt_s[1] < K1))
                            def _():
                                append(d <= r1sq, 1, K1, buf1_v, pidxf)

                            @pl.when((dmins[q] <= r0sq) & (cnt_s[0] < K0))
                            def _():
                                append(d <= r0sq, 0, K0, buf0_v, pidxf)

            return 0

        lax.fori_loop(0, n // SUPER, outer, 0)

        c0f = cnt_s[0]
        c1f = cnt_s[1]
        v0 = buf0_v[pl.ds(0, 16)].astype(jnp.int32)
        first0 = v0[0]
        idx0_v[pl.ds(0, 16)] = jnp.where(io < c0f, v0, first0)
        first1 = buf1_v[pl.ds(0, 16)].astype(jnp.int32)[0]
        for h in range(2):
            vh = buf1_v[pl.ds(h * 16, 16)].astype(jnp.int32)
            gi = (h * 16) + io
            idx1_v[pl.ds(h * 16, 16)] = jnp.where(gi < c1f, vh, first1)

        pltpu.async_copy(pc_hbm.at[idx0_v], rows0_v, sem).wait()
        pltpu.async_copy(pc_hbm.at[idx1_v], rows1_v, sem).wait()
        pltpu.sync_copy(rows0_v, out0_hbm.at[bi, s_base + j])
        pltpu.sync_copy(rows1_v, out1_hbm.at[bi, s_base + j])
        return 0

    lax.fori_loop(0, 8, center_body, 0)


def _sc_ball(xyzt, pc_flat, cidx_adj):
    bsz, _, n = xyzt.shape
    mesh = plsc.VectorSubcoreMesh(core_axis_name="c", subcore_axis_name="s",
                                  num_cores=NC, num_subcores=NS)
    f = pl.kernel(
        _sc_ball_body,
        out_type=(jax.ShapeDtypeStruct((bsz, NPATCH, K0, RW), jnp.float32),
                  jax.ShapeDtypeStruct((bsz, NPATCH, K1, RW), jnp.float32),
                  jax.ShapeDtypeStruct((bsz, NPATCH, RW), jnp.float32)),
        mesh=mesh,
        scratch_types=[
            pltpu.VMEM((3, n), jnp.float32),
            pltpu.VMEM((8,), jnp.int32),
            pltpu.VMEM((8, RW), jnp.float32),
            pltpu.VMEM((K0,), jnp.int32),
            pltpu.VMEM((K1,), jnp.int32),
            pltpu.VMEM((K0 + 16,), jnp.float32),
            pltpu.VMEM((K1 + 16,), jnp.float32),
            pltpu.VMEM((K0, RW), jnp.float32),
            pltpu.VMEM((K1, RW), jnp.float32),
            pltpu.SMEM((4,), jnp.int32),
            pltpu.SemaphoreType.DMA,
        ],
        compiler_params=pltpu.CompilerParams(use_tc_tiling_on_sc=False),
    )
    return f(xyzt, pc_flat, cidx_adj)


def _scale_body(x_ref, c_ref, w1, b1, g1, be1, w2, b2, g2, be2,
                e24_ref, wgh, wgr, wgp, bg, gg, beg, wh, bh, gh, beh,
                out_ref, *, k, inv_r):
    x = x_ref[...]
    h = _mlp(x, w1[...], b1[...], g1[...], be1[...], True)
    h = _mlp(h, w2[...], b2[...], g2[...], be2[...], True)
    r8 = x.shape[0]
    colm = (lax.broadcasted_iota(jnp.int32, (1, 8), 1) < 3).astype(jnp.float32)
    relp = (x[:, 0:8] - c_ref[...][:, 0:8]) * jnp.float32(inv_r) * colm
    args = _dot(relp, e24_ref[...])
    sel = (lax.broadcasted_iota(jnp.int32, (r8, PE_REL), 1) % 8) < 4
    pe = jnp.where(sel, jnp.sin(args), jnp.cos(args))
    z = _dot(h, wgh[...]) + _dot(relp, wgr[...]) + _dot(pe, wgp[...]) + bg[...]
    z = _gelu(_ln(z, gg[...], beg[...]))
    z = _mlp(z, wh[...], bh[...], gh[...], beh[...], False)
    grp = z.reshape(r8 // k, k, TOK)
    out_ref[...] = jnp.max(grp, axis=1)


def _scale(rows, crep, p_stem, enc, k, radius):
    r8 = rows.shape[0]
    w1 = jnp.zeros((RW, STEM), jnp.float32).at[:CIN].set(p_stem[0]["W"])
    wg = enc[0]["W"]
    wgh = wg[:STEM]
    wgr = jnp.zeros((8, TOK), jnp.float32).at[:3].set(wg[STEM:STEM + 3])
    wgp = wg[STEM + 3:]
    freqs = (2.0 ** jnp.arange(4, dtype=jnp.float32)) * jnp.pi
    e24 = jnp.zeros((8, PE_REL), jnp.float32)
    for c in range(3):
        e24 = e24.at[c, c * 8:c * 8 + 4].set(freqs)
        e24 = e24.at[c, c * 8 + 4:c * 8 + 8].set(freqs)
    row = lambda a: a.reshape(1, -1)
    body = functools.partial(_scale_body, k=k, inv_r=1.0 / max(radius, 1e-6))
    return pl.pallas_call(
        body,
        out_shape=jax.ShapeDtypeStruct((r8 // k, TOK), jnp.float32),
    )(rows, crep,
      w1, row(p_stem[0]["b"]), row(p_stem[0]["g"]), row(p_stem[0]["beta"]),
      p_stem[1]["W"], row(p_stem[1]["b"]), row(p_stem[1]["g"]), row(p_stem[1]["beta"]),
      e24, wgh, wgr, wgp,
      row(enc[0]["b"]), row(enc[0]["g"]), row(enc[0]["beta"]),
      enc[1]["W"], row(enc[1]["b"]), row(enc[1]["g"]), row(enc[1]["beta"]))


def _proj_body(c_ref, s0_ref, s1_ref, w1, b1, g1, be1, w2, b2, g2, be2,
               e96_ref, wpc, wp0, wp1, wpe, bp, gp, bep,
               wq, bq, gq, beq, out_ref):
    x = c_ref[...]
    cf = _mlp(x, w1[...], b1[...], g1[...], be1[...], True)
    cf = _mlp(cf, w2[...], b2[...], g2[...], be2[...], True)
    r8 = x.shape[0]
    colm = (lax.broadcasted_iota(jnp.int32, (1, 8), 1) < 3).astype(jnp.float32)
    cxyz = x[:, 0:8] * colm
    args = _dot(cxyz, e96_ref[...])
    sel = (lax.broadcasted_iota(jnp.int32, (r8, PE_GLB), 1) % 32) < 16
    cpe = jnp.where(sel, jnp.sin(args), jnp.cos(args))
    z = (_dot(cf, wpc[...]) + _dot(s0_ref[...], wp0[...])
         + _dot(s1_ref[...], wp1[...]) + _dot(cpe, wpe[...]) + bp[...])
    z = _gelu(_ln(z, gp[...], bep[...]))
    z = _mlp(z, wq[...], bq[...], gq[...], beq[...], False)
    out_ref[...] = z


def _proj(rows_c, s0, s1, p_stem, proj):
    bs = rows_c.shape[0]
    w1 = jnp.zeros((RW, STEM), jnp.float32).at[:CIN].set(p_stem[0]["W"])
    wp = proj[0]["W"]
    wpc = wp[:STEM]
    wp0 = wp[STEM:STEM + TOK]
    wp1 = wp[STEM + TOK:STEM + 2 * TOK]
    wpe = wp[STEM + 2 * TOK:]
    freqs = (2.0 ** jnp.arange(16, dtype=jnp.float32)) * jnp.pi
    e96 = jnp.zeros((8, PE_GLB), jnp.float32)
    for c in range(3):
        e96 = e96.at[c, c * 32:c * 32 + 16].set(freqs)
        e96 = e96.at[c, c * 32 + 16:c * 32 + 32].set(freqs)
    row = lambda a: a.reshape(1, -1)
    return pl.pallas_call(
        _proj_body,
        out_shape=jax.ShapeDtypeStruct((bs, TOK), jnp.float32),
    )(rows_c, s0, s1,
      w1, row(p_stem[0]["b"]), row(p_stem[0]["g"]), row(p_stem[0]["beta"]),
      p_stem[1]["W"], row(p_stem[1]["b"]), row(p_stem[1]["g"]), row(p_stem[1]["beta"]),
      e96, wpc, wp0, wp1, wpe,
      row(proj[0]["b"]), row(proj[0]["g"]), row(proj[0]["beta"]),
      proj[1]["W"], row(proj[1]["b"]), row(proj[1]["g"]), row(proj[1]["beta"]))


def kernel(pointcloud, params):
    b, n, cin = pointcloud.shape
    xyzt = jnp.transpose(pointcloud[..., :3], (0, 2, 1))
    cidx = _fps(xyzt[:, 0], xyzt[:, 1], xyzt[:, 2])
    pc_flat = jnp.pad(pointcloud, ((0, 0), (0, 0), (0, RW - cin)))
    pc_flat = pc_flat.reshape(b * n, RW)
    cidx_adj = cidx + (jnp.arange(b, dtype=jnp.int32) * n)[:, None]
    rows0, rows1, rows_c = _sc_ball(xyzt, pc_flat, cidx_adj)
    centers = rows_c[..., :3]
    bs = b * NPATCH
    crep = rows_c.reshape(b, NPATCH, 1, RW)
    s0 = _scale(rows0.reshape(bs * K0, RW),
                jnp.broadcast_to(crep, (b, NPATCH, K0, RW)).reshape(bs * K0, RW),
                params["stem"], params["enc"][0], K0, RADII[0])
    s1 = _scale(rows1.reshape(bs * K1, RW),
                jnp.broadcast_to(crep, (b, NPATCH, K1, RW)).reshape(bs * K1, RW),
                params["stem"], params["enc"][1], K1, RADII[1])
    t = _proj(rows_c.reshape(bs, RW), s0, s1, params["stem"], params["proj"])
    return t.reshape(b, NPATCH, TOK), centers
